# Initial kernel scaffold; baseline (speedup 1.0000x reference)
import jax
import jax.numpy as jnp
from jax.experimental import pallas as pl


def kernel(x_gene, x_disease, x_drug, params, ei_gene_interacts_gene, ei_gene_associated_disease, ei_drug_targets_gene):
    raise NotImplementedError("write your pallas kernel here")


# speedup vs baseline: 11.7590x; 11.7590x over previous
import functools

import jax
import jax.numpy as jnp
import numpy as np
from jax import lax
from jax.experimental import pallas as pl
from jax.experimental.pallas import tpu as pltpu
from jax.experimental.pallas import tpu_sc as plsc

_NODE_TYPES = ("gene", "disease", "drug")
_EDGE_TYPES = (("gene", "interacts", "gene"),
               ("gene", "associated", "disease"),
               ("drug", "targets", "gene"))
_COUNTS = {"gene": 25000, "disease": 15000, "drug": 10000}
_HID = 128
_HEADS = 8
_DH = 16


def _rk(et):
    return "__".join(et)


def _mm_body(x_ref, w_ref, b_ref, o_ref):
    o_ref[...] = (jnp.dot(x_ref[...], w_ref[...],
                          preferred_element_type=jnp.float32) + b_ref[...])


def _matmul_bias(x, w, b, block=1000):
    n, k = x.shape
    c = w.shape[1]
    assert n % block == 0
    return pl.pallas_call(
        _mm_body,
        grid=(n // block,),
        in_specs=[pl.BlockSpec((block, k), lambda i: (i, 0)),
                  pl.BlockSpec((k, c), lambda i: (0, 0)),
                  pl.BlockSpec((1, c), lambda i: (0, 0))],
        out_specs=pl.BlockSpec((block, c), lambda i: (i, 0)),
        out_shape=jax.ShapeDtypeStruct((n, c), jnp.float32),
    )(x, w, b.reshape(1, c))


def _logits_body(q_ref, k_ref, s_ref, l_ref, m_ref):
    i = pl.program_id(0)
    l = jnp.dot(q_ref[...] * k_ref[...], s_ref[...],
                preferred_element_type=jnp.float32)
    l_ref[...] = l
    bm = jnp.max(l, axis=0, keepdims=True)

    @pl.when(i == 0)
    def _():
        m_ref[...] = bm

    @pl.when(i > 0)
    def _():
        m_ref[...] = jnp.maximum(m_ref[...], bm)


def _edge_logits(qe, ke, s, block):
    e = qe.shape[0]
    assert e % block == 0
    return pl.pallas_call(
        _logits_body,
        grid=(e // block,),
        in_specs=[pl.BlockSpec((block, _HID), lambda i: (i, 0)),
                  pl.BlockSpec((block, _HID), lambda i: (i, 0)),
                  pl.BlockSpec((_HID, 16), lambda i: (0, 0))],
        out_specs=[pl.BlockSpec((block, 16), lambda i: (i, 0)),
                   pl.BlockSpec((1, 16), lambda i: (0, 0))],
        out_shape=[jax.ShapeDtypeStruct((e, 16), jnp.float32),
                   jax.ShapeDtypeStruct((1, 16), jnp.float32)],
    )(qe, ke, s)


def _expmsg_body(l_ref, m_ref, v_ref, st_ref, e_ref, msg_ref):
    ev = jnp.exp(l_ref[...] - m_ref[...])
    e_ref[...] = ev
    msg_ref[...] = v_ref[...] * jnp.dot(ev, st_ref[...],
                                        preferred_element_type=jnp.float32)


def _exp_and_msg(logits, gmax, ve, st, block):
    e = logits.shape[0]
    assert e % block == 0
    return pl.pallas_call(
        _expmsg_body,
        grid=(e // block,),
        in_specs=[pl.BlockSpec((block, 16), lambda i: (i, 0)),
                  pl.BlockSpec((1, 16), lambda i: (0, 0)),
                  pl.BlockSpec((block, _HID), lambda i: (i, 0)),
                  pl.BlockSpec((16, _HID), lambda i: (0, 0))],
        out_specs=[pl.BlockSpec((block, 16), lambda i: (i, 0)),
                   pl.BlockSpec((block, _HID), lambda i: (i, 0))],
        out_shape=[jax.ShapeDtypeStruct((e, 16), jnp.float32),
                   jax.ShapeDtypeStruct((e, _HID), jnp.float32)],
    )(logits, gmax, ve, st)


def _out_body(nrel, x_ref, *refs):
    msgs = refs[0:2 * nrel:2]
    sums = refs[1:2 * nrel:2]
    aw_ref, ab_ref, st_ref, beta_ref, o_ref = refs[2 * nrel:]
    agg = jnp.zeros_like(x_ref[...])
    for m_ref, s_ref in zip(msgs, sums):
        rep = jnp.dot(s_ref[...], st_ref[...],
                      preferred_element_type=jnp.float32)
        agg = agg + m_ref[...] / (rep + 1e-30)
    o = jax.nn.gelu(agg)
    o = jnp.dot(o, aw_ref[...], preferred_element_type=jnp.float32) + ab_ref[...]
    beta = beta_ref[...]
    o = beta * o + (1.0 - beta) * x_ref[...]
    o_ref[...] = jnp.maximum(o, 0.0) + x_ref[...]


def _out_stage(x, pairs, aw, ab, st, beta, block=1000):
    n = x.shape[0]
    assert n % block == 0
    nrel = len(pairs)
    in_specs = [pl.BlockSpec((block, _HID), lambda i: (i, 0))]
    args = [x]
    for msum, ssum in pairs:
        in_specs.append(pl.BlockSpec((block, _HID), lambda i: (i, 0)))
        in_specs.append(pl.BlockSpec((block, 16), lambda i: (i, 0)))
        args += [msum, ssum]
    in_specs += [pl.BlockSpec((_HID, _HID), lambda i: (0, 0)),
                 pl.BlockSpec((1, _HID), lambda i: (0, 0)),
                 pl.BlockSpec((16, _HID), lambda i: (0, 0)),
                 pl.BlockSpec((1, _HID), lambda i: (0, 0))]
    args += [aw, ab.reshape(1, _HID), st, beta]
    return pl.pallas_call(
        functools.partial(_out_body, nrel),
        grid=(n // block,),
        in_specs=in_specs,
        out_specs=pl.BlockSpec((block, _HID), lambda i: (i, 0)),
        out_shape=jax.ShapeDtypeStruct((n, _HID), jnp.float32),
    )(*args)


def _gather_rows(qtab, ktab, vtab, src, dst):
    return qtab[dst], ktab[src], vtab[src]


def _scatter_rows(msgu, expv, dst, ndst):
    msum = jax.ops.segment_sum(msgu, dst, num_segments=ndst)
    ssum = jax.ops.segment_sum(expv, dst, num_segments=ndst)
    return msum, ssum


def _fold_rel(w, b, a):
    w3 = w.reshape(_HID, _HEADS, _DH)
    wf = jnp.einsum("khd,hde->khe", w3, a).reshape(_HID, _HID)
    bf = jnp.einsum("hd,hde->he", b.reshape(_HEADS, _DH), a).reshape(_HID)
    return wf, bf


def _head_sum_matrix(scale):
    m = np.zeros((_HID, 16), np.float32)
    for h in range(_HEADS):
        m[h * _DH:(h + 1) * _DH, h] = 1.0
    return jnp.asarray(m) * scale.reshape(1, 16)


_ST = jnp.asarray(
    np.concatenate([np.kron(np.eye(_HEADS, dtype=np.float32),
                            np.ones((1, _DH), np.float32)),
                    np.zeros((_HEADS, _HID), np.float32)], axis=0))


def _layer(xd, eid, lp):
    counts = _COUNTS
    need = {t: [("q", lp["q_w"][t], lp["q_b"][t])] if t != "drug" else []
            for t in _NODE_TYPES}
    for et in _EDGE_TYPES:
        s_t, _, _ = et
        r = _rk(et)
        kw, kb = _fold_rel(lp["k_w"][s_t], lp["k_b"][s_t], lp["a_rel"][r])
        vw, vb = _fold_rel(lp["v_w"][s_t], lp["v_b"][s_t], lp["m_rel"][r])
        need[s_t].append(("k:" + r, kw, kb))
        need[s_t].append(("v:" + r, vw, vb))
    proj = {}
    for t in _NODE_TYPES:
        names = [n for n, _, _ in need[t]]
        wcat = jnp.concatenate([w for _, w, _ in need[t]], axis=1)
        bcat = jnp.concatenate([b for _, _, b in need[t]], axis=0)
        y = _matmul_bias(xd[t], wcat, bcat)
        for j, n in enumerate(names):
            proj[(t, n)] = y[:, j * _HID:(j + 1) * _HID]

    pairs = {t: [] for t in _NODE_TYPES}
    for et, block in zip(_EDGE_TYPES, (1000, 1000, 1000)):
        s_t, _, d_t = et
        r = _rk(et)
        src, dst = eid[r][0], eid[r][1]
        qe, ke, ve = _gather_rows(proj[(d_t, "q")], proj[(s_t, "k:" + r)],
                                  proj[(s_t, "v:" + r)], src, dst)
        scale = jnp.concatenate([lp["p_rel"][r] / np.sqrt(_DH),
                                 jnp.zeros((8,), jnp.float32)])
        smat = _head_sum_matrix(scale)
        logits, gmax = _edge_logits(qe, ke, smat, block)
        expv, msgu = _exp_and_msg(logits, gmax, ve, _ST, block)
        msum, ssum = _scatter_rows(msgu, expv, dst, counts[d_t])
        pairs[d_t].append((msum, ssum))

    out = {}
    for t in _NODE_TYPES:
        beta = jax.nn.sigmoid(lp["skip"][t])
        beta_arr = jnp.full((1, _HID), beta, jnp.float32)
        out[t] = _out_stage(xd[t], pairs[t], lp["a_w"][t], lp["a_b"][t],
                            _ST, beta_arr)
    return out


def kernel(x_gene, x_disease, x_drug, params,
           ei_gene_interacts_gene, ei_gene_associated_disease,
           ei_drug_targets_gene):
    eid = {
        "gene__interacts__gene": ei_gene_interacts_gene,
        "gene__associated__disease": ei_gene_associated_disease,
        "drug__targets__gene": ei_drug_targets_gene,
    }
    xd = {"gene": x_gene, "disease": x_disease, "drug": x_drug}
    for lp in params:
        xd = _layer(xd, eid, lp)
    return (xd["gene"], xd["disease"], xd["drug"])


# speedup vs baseline: 20.9074x; 1.7780x over previous
import functools

import jax
import jax.numpy as jnp
import numpy as np
from jax import lax
from jax.experimental import pallas as pl
from jax.experimental.pallas import tpu as pltpu
from jax.experimental.pallas import tpu_sc as plsc

_NODE_TYPES = ("gene", "disease", "drug")
_EDGE_TYPES = (("gene", "interacts", "gene"),
               ("gene", "associated", "disease"),
               ("drug", "targets", "gene"))
_COUNTS = {"gene": 25000, "disease": 15000, "drug": 10000}
_HID = 128
_HEADS = 8
_DH = 16


def _rk(et):
    return "__".join(et)


def _mm_body(x_ref, w_ref, b_ref, o_ref):
    o_ref[...] = (jnp.dot(x_ref[...], w_ref[...],
                          preferred_element_type=jnp.float32) + b_ref[...])


def _matmul_bias(x, w, b, block=1000):
    n, k = x.shape
    c = w.shape[1]
    assert n % block == 0
    return pl.pallas_call(
        _mm_body,
        grid=(n // block,),
        in_specs=[pl.BlockSpec((block, k), lambda i: (i, 0)),
                  pl.BlockSpec((k, c), lambda i: (0, 0)),
                  pl.BlockSpec((1, c), lambda i: (0, 0))],
        out_specs=pl.BlockSpec((block, c), lambda i: (i, 0)),
        out_shape=jax.ShapeDtypeStruct((n, c), jnp.float32),
    )(x, w, b.reshape(1, c))


def _logits_body(q_ref, k_ref, s_ref, l_ref, m_ref):
    i = pl.program_id(0)
    l = jnp.dot(q_ref[...] * k_ref[...], s_ref[...],
                preferred_element_type=jnp.float32)
    l_ref[...] = l
    bm = jnp.max(l, axis=0, keepdims=True)

    @pl.when(i == 0)
    def _():
        m_ref[...] = bm

    @pl.when(i > 0)
    def _():
        m_ref[...] = jnp.maximum(m_ref[...], bm)


def _edge_logits(qe, ke, s, block):
    e = qe.shape[0]
    assert e % block == 0
    return pl.pallas_call(
        _logits_body,
        grid=(e // block,),
        in_specs=[pl.BlockSpec((block, _HID), lambda i: (i, 0)),
                  pl.BlockSpec((block, _HID), lambda i: (i, 0)),
                  pl.BlockSpec((_HID, 16), lambda i: (0, 0))],
        out_specs=[pl.BlockSpec((block, 16), lambda i: (i, 0)),
                   pl.BlockSpec((1, 16), lambda i: (0, 0))],
        out_shape=[jax.ShapeDtypeStruct((e, 16), jnp.float32),
                   jax.ShapeDtypeStruct((1, 16), jnp.float32)],
    )(qe, ke, s)


def _expmsg_body(l_ref, m_ref, v_ref, st_ref, e_ref, msg_ref):
    ev = jnp.exp(l_ref[...] - m_ref[...])
    e_ref[...] = ev
    msg_ref[...] = v_ref[...] * jnp.dot(ev, st_ref[...],
                                        preferred_element_type=jnp.float32)


def _exp_and_msg(logits, gmax, ve, st, block):
    e = logits.shape[0]
    assert e % block == 0
    return pl.pallas_call(
        _expmsg_body,
        grid=(e // block,),
        in_specs=[pl.BlockSpec((block, 16), lambda i: (i, 0)),
                  pl.BlockSpec((1, 16), lambda i: (0, 0)),
                  pl.BlockSpec((block, _HID), lambda i: (i, 0)),
                  pl.BlockSpec((16, _HID), lambda i: (0, 0))],
        out_specs=[pl.BlockSpec((block, 16), lambda i: (i, 0)),
                   pl.BlockSpec((block, _HID), lambda i: (i, 0))],
        out_shape=[jax.ShapeDtypeStruct((e, 16), jnp.float32),
                   jax.ShapeDtypeStruct((e, _HID), jnp.float32)],
    )(logits, gmax, ve, st)


def _out_body(nrel, x_ref, *refs):
    msgs = refs[0:2 * nrel:2]
    sums = refs[1:2 * nrel:2]
    aw_ref, ab_ref, st_ref, beta_ref, o_ref = refs[2 * nrel:]
    agg = jnp.zeros_like(x_ref[...])
    for m_ref, s_ref in zip(msgs, sums):
        rep = jnp.dot(s_ref[...], st_ref[...],
                      preferred_element_type=jnp.float32)
        agg = agg + m_ref[...] / (rep + 1e-30)
    o = jax.nn.gelu(agg)
    o = jnp.dot(o, aw_ref[...], preferred_element_type=jnp.float32) + ab_ref[...]
    beta = beta_ref[...]
    o = beta * o + (1.0 - beta) * x_ref[...]
    o_ref[...] = jnp.maximum(o, 0.0) + x_ref[...]


def _out_stage(x, pairs, aw, ab, st, beta, block=1000):
    n = x.shape[0]
    assert n % block == 0
    nrel = len(pairs)
    in_specs = [pl.BlockSpec((block, _HID), lambda i: (i, 0))]
    args = [x]
    for msum, ssum in pairs:
        in_specs.append(pl.BlockSpec((block, _HID), lambda i: (i, 0)))
        in_specs.append(pl.BlockSpec((block, 16), lambda i: (i, 0)))
        args += [msum, ssum]
    in_specs += [pl.BlockSpec((_HID, _HID), lambda i: (0, 0)),
                 pl.BlockSpec((1, _HID), lambda i: (0, 0)),
                 pl.BlockSpec((16, _HID), lambda i: (0, 0)),
                 pl.BlockSpec((1, _HID), lambda i: (0, 0))]
    args += [aw, ab.reshape(1, _HID), st, beta]
    return pl.pallas_call(
        functools.partial(_out_body, nrel),
        grid=(n // block,),
        in_specs=in_specs,
        out_specs=pl.BlockSpec((block, _HID), lambda i: (i, 0)),
        out_shape=jax.ShapeDtypeStruct((n, _HID), jnp.float32),
    )(*args)


_SC_MESH = plsc.VectorSubcoreMesh(core_axis_name="c", subcore_axis_name="s")


def _round_up(x, m):
    return (x + m - 1) // m * m


def _sc_gather_all(tabs):
    nr = len(tabs)
    nits = [epad // 4096 for (_, _, _, _, _, epad) in tabs]
    nmax = max(nits)

    def body(*refs):
        ins = refs[:nr * 5]
        outs = refs[nr * 5:nr * 8]
        idxs, idxd, rq, rk, rv, sq, sk, sv = refs[nr * 8:]
        wid = lax.axis_index("s") * 2 + lax.axis_index("c")
        for t in range(nr):
            q_hbm, k_hbm, v_hbm, si_hbm, di_hbm = ins[t * 5:(t + 1) * 5]
            qe, ke, ve = outs[t * 3:(t + 1) * 3]
            nit = nits[t]
            r0 = wid * nit
            pltpu.sync_copy(si_hbm.at[pl.ds(r0, nit)],
                            idxs.at[pl.ds(0, nit)])
            pltpu.sync_copy(di_hbm.at[pl.ds(r0, nit)],
                            idxd.at[pl.ds(0, nit)])

            def step(m, carry, q_hbm=q_hbm, k_hbm=k_hbm, v_hbm=v_hbm,
                     qe=qe, ke=ke, ve=ve, r0=r0):
                e0 = (r0 + m) * 128
                cq = pltpu.async_copy(q_hbm.at[idxd.at[m, 0]], rq, sq)
                ck = pltpu.async_copy(k_hbm.at[idxs.at[m, 0]], rk, sk)
                cv = pltpu.async_copy(v_hbm.at[idxs.at[m, 0]], rv, sv)
                cq.wait()
                ck.wait()
                cv.wait()
                pltpu.sync_copy(rq, qe.at[pl.ds(e0, 128), :])
                pltpu.sync_copy(rk, ke.at[pl.ds(e0, 128), :])
                pltpu.sync_copy(rv, ve.at[pl.ds(e0, 128), :])
                return carry

            lax.fori_loop(0, nit, step, 0)

    args = []
    out_type = []
    for (qtab, ktab, vtab, si2d, di2d, epad) in tabs:
        args += [qtab, ktab, vtab, si2d, di2d]
        out_type += [jax.ShapeDtypeStruct((epad, _HID), jnp.float32)] * 3
    outs = pl.kernel(
        body,
        out_type=out_type,
        mesh=_SC_MESH,
        scratch_types=[
            pltpu.VMEM((nmax, 1, 128), jnp.int32),
            pltpu.VMEM((nmax, 1, 128), jnp.int32),
            pltpu.VMEM((128, _HID), jnp.float32),
            pltpu.VMEM((128, _HID), jnp.float32),
            pltpu.VMEM((128, _HID), jnp.float32),
            pltpu.SemaphoreType.DMA,
            pltpu.SemaphoreType.DMA,
            pltpu.SemaphoreType.DMA,
        ],
    )(*args)
    return [tuple(outs[t * 3:(t + 1) * 3]) for t in range(nr)]


def _fold_rel(w, b, a):
    w3 = w.reshape(_HID, _HEADS, _DH)
    wf = jnp.einsum("khd,hde->khe", w3, a).reshape(_HID, _HID)
    bf = jnp.einsum("hd,hde->he", b.reshape(_HEADS, _DH), a).reshape(_HID)
    return wf, bf


def _head_sum_np():
    m = np.zeros((_HID, 16), np.float32)
    for h in range(_HEADS):
        m[h * _DH:(h + 1) * _DH, h] = 1.0
    return m


_SM_NP = _head_sum_np()
_ST = np.concatenate([np.kron(np.eye(_HEADS, dtype=np.float32),
                              np.ones((1, _DH), np.float32)),
                      np.zeros((_HEADS, _HID), np.float32)], axis=0)


def _layer(xd, eid, lp):
    counts = _COUNTS
    need = {t: [("q", lp["q_w"][t], lp["q_b"][t])] if t != "drug" else []
            for t in _NODE_TYPES}
    for et in _EDGE_TYPES:
        s_t, _, _ = et
        r = _rk(et)
        kw, kb = _fold_rel(lp["k_w"][s_t], lp["k_b"][s_t], lp["a_rel"][r])
        vw, vb = _fold_rel(lp["v_w"][s_t], lp["v_b"][s_t], lp["m_rel"][r])
        need[s_t].append(("k:" + r, kw, kb))
        need[s_t].append(("v:" + r, vw, vb))
    proj = {}
    for t in _NODE_TYPES:
        names = [n for n, _, _ in need[t]]
        wcat = jnp.concatenate([w for _, w, _ in need[t]], axis=1)
        bcat = jnp.concatenate([b for _, _, b in need[t]], axis=0)
        y = _matmul_bias(xd[t], wcat, bcat)
        for j, n in enumerate(names):
            proj[(t, n)] = y[:, j * _HID:(j + 1) * _HID]

    gtabs = []
    for et in _EDGE_TYPES:
        s_t, _, d_t = et
        r = _rk(et)
        si2d, dig2d, _, epad = eid[r]
        gtabs.append((proj[(d_t, "q")], proj[(s_t, "k:" + r)],
                      proj[(s_t, "v:" + r)], si2d, dig2d, epad))
    gath = _sc_gather_all(gtabs)

    jobs = []
    for et, (qe, ke, ve) in zip(_EDGE_TYPES, gath):
        r = _rk(et)
        _, _, dis2d, epad = eid[r]
        scale = jnp.concatenate([lp["p_rel"][r] / np.sqrt(_DH),
                                 jnp.zeros((8,), jnp.float32)])
        smat = jnp.asarray(_SM_NP) * scale.reshape(1, 16)
        logits, gmax = _edge_logits(qe, ke, smat, 2048)
        expv, msgu = _exp_and_msg(logits, gmax, ve, jnp.asarray(_ST), 2048)
        jobs.append((msgu, expv, dis2d, counts[et[2]], epad))

    pairs = {t: [] for t in _NODE_TYPES}
    for et, (msgu, expv, dis2d, nd, epad) in zip(_EDGE_TYPES, jobs):
        dstf = dis2d.reshape(-1)
        msum = jax.ops.segment_sum(msgu, dstf, num_segments=nd + 1)[:nd]
        ssum = jax.ops.segment_sum(expv, dstf, num_segments=nd + 1)[:nd]
        pairs[et[2]].append((msum, ssum))

    out = {}
    for t in _NODE_TYPES:
        beta = jax.nn.sigmoid(lp["skip"][t])
        beta_arr = jnp.full((1, _HID), beta, jnp.float32)
        out[t] = _out_stage(xd[t], pairs[t], lp["a_w"][t], lp["a_b"][t],
                            _ST, beta_arr)
    return out


def _prep_edges(ei, nd):
    e = ei.shape[1]
    epad = _round_up(e, 4096)
    src, dst = ei[0], ei[1]
    pads = epad - e
    src2 = jnp.concatenate([src, jnp.zeros((pads,), jnp.int32)])
    dstg = jnp.concatenate([dst, jnp.zeros((pads,), jnp.int32)])
    dsts = jnp.concatenate([dst, jnp.full((pads,), nd, jnp.int32)])
    return (src2.reshape(-1, 1, 128), dstg.reshape(-1, 1, 128),
            dsts.reshape(-1, 1, 128), epad)


def kernel(x_gene, x_disease, x_drug, params,
           ei_gene_interacts_gene, ei_gene_associated_disease,
           ei_drug_targets_gene):
    eis = {
        "gene__interacts__gene": ei_gene_interacts_gene,
        "gene__associated__disease": ei_gene_associated_disease,
        "drug__targets__gene": ei_drug_targets_gene,
    }
    eid = {_rk(et): _prep_edges(eis[_rk(et)], _COUNTS[et[2]])
           for et in _EDGE_TYPES}
    xd = {"gene": x_gene, "disease": x_disease, "drug": x_drug}
    for lp in params:
        xd = _layer(xd, eid, lp)
    return (xd["gene"], xd["disease"], xd["drug"])
